# baseline (speedup 1.0000x reference)
import math

import jax
import jax.numpy as jnp
from jax.experimental import pallas as pl
from jax.experimental.pallas import tpu as pltpu

L = 2048
C = 16 * 64
T = 256
NB = L // T
ND = NB // 2 + 1
TOP_K = max(1, int(3 * math.log(L)))
NEG_INF = float("-inf")


def _gram_kernel(q_ref, d_ref):
    d = pl.program_id(1)
    acc = jnp.zeros((T, T), jnp.float32)
    for a in range(NB):
        qa = q_ref[0, pl.ds(a * T, T), :]
        ab = ((a + d) % NB) * T
        qb = q_ref[0, pl.ds(pl.multiple_of(ab, T), T), :]
        acc += jax.lax.dot_general(
            qa, qb,
            dimension_numbers=(((1,), (1,)), ((), ())),
            preferred_element_type=jnp.float32,
        )
    d_ref[0, 0] = acc


def _gram(q):
    return pl.pallas_call(
        _gram_kernel,
        grid=(2, ND),
        in_specs=[
            pl.BlockSpec((1, L, C), lambda b, d: (b, 0, 0)),
        ],
        out_specs=pl.BlockSpec((1, 1, T, T), lambda b, d: (b, d, 0, 0)),
        out_shape=jax.ShapeDtypeStruct((2, ND, T, T), jnp.float32),
    )(q)


def _topk_kernel(d_ref, idx_ref, w_ref):
    d04 = d_ref[...]
    parts = [d04] + [
        jnp.transpose(d04[:, k], (0, 2, 1)).reshape(2, 1, T, T)
        for k in (3, 2, 1)
    ]
    D = jnp.concatenate(parts, axis=1)
    E = jnp.concatenate([D, jnp.zeros_like(D)], axis=-1)
    u = jax.lax.broadcasted_iota(jnp.int32, E.shape, 2)
    for j in range(8):
        sh = 1 << j
        rolled = jnp.concatenate([E[..., sh:], E[..., :sh]], axis=-1)
        E = jnp.where((u & sh) != 0, rolled, E)
    corrp = jnp.sum(E, axis=2)
    nxt = jnp.roll(corrp, shift=-1, axis=1)
    mc = (corrp[:, :, :T] + nxt[:, :, T:]) * (1.0 / C)

    m = 0.5 * (mc[0] + mc[1])
    fi = (jax.lax.broadcasted_iota(jnp.int32, (NB, T), 0) * T
          + jax.lax.broadcasted_iota(jnp.int32, (NB, T), 1))
    lane = jax.lax.broadcasted_iota(jnp.int32, (8, 128), 1)
    row = jax.lax.broadcasted_iota(jnp.int32, (8, 128), 0)

    def body(i, carry):
        m, ivec, svec = carry
        val = jnp.max(m)
        idx = jnp.min(jnp.where(m == val, fi, jnp.int32(2 * L)))
        hit = fi == idx
        s0 = jnp.sum(jnp.where(hit, mc[0], 0.0))
        s1 = jnp.sum(jnp.where(hit, mc[1], 0.0))
        here = lane == i
        ivec = jnp.where((row == 0) & here, idx, ivec)
        svec = jnp.where((row == 0) & here, s0, svec)
        svec = jnp.where((row == 1) & here, s1, svec)
        m = jnp.where(hit, NEG_INF, m)
        return m, ivec, svec

    ivec = jnp.zeros((8, 128), jnp.int32)
    svec = jnp.zeros((8, 128), jnp.float32)
    m, ivec, svec = jax.lax.fori_loop(0, TOP_K, body, (m, ivec, svec))

    valid = lane < TOP_K
    x = jnp.where(valid, svec, NEG_INF)
    xmax = jnp.max(x, axis=1, keepdims=True)
    ex = jnp.where(valid, jnp.exp(x - xmax), 0.0)
    w = ex / jnp.sum(ex, axis=1, keepdims=True)

    idx_ref[...] = ivec
    w_ref[...] = w


def _topk(d_mats):
    return pl.pallas_call(
        _topk_kernel,
        out_shape=(
            jax.ShapeDtypeStruct((8, 128), jnp.int32),
            jax.ShapeDtypeStruct((8, 128), jnp.float32),
        ),
    )(d_mats)


def _agg_kernel(idx_ref, w_ref, v_ref, out_ref):
    b = pl.program_id(0)
    d0 = idx_ref[0]

    @pl.when(d0 == 0)
    def _():
        out_ref[0] = v_ref[0] * w_ref[b, 0]

    @pl.when(d0 != 0)
    def _():
        out_ref[0] = pltpu.roll(v_ref[0], -d0, axis=0) * w_ref[b, 0]

    for i in range(1, TOP_K):
        w = w_ref[b, i]

        @pl.when(w != 0.0)
        def _(i=i, w=w):
            out_ref[0] += pltpu.roll(v_ref[0], -idx_ref[i], axis=0) * w


def _aggregate(idx, w, v, cb=256):
    return pl.pallas_call(
        _agg_kernel,
        grid=(2, C // cb),
        in_specs=[
            pl.BlockSpec(memory_space=pltpu.SMEM),
            pl.BlockSpec(memory_space=pltpu.SMEM),
            pl.BlockSpec((1, L, cb), lambda b, c: (b, 0, c)),
        ],
        out_specs=pl.BlockSpec((1, L, cb), lambda b, c: (b, 0, c)),
        out_shape=jax.ShapeDtypeStruct((2, L, C), jnp.float32),
    )(idx, w, v)


@jax.jit
def kernel(qk, values):
    B, Lx, H, E = qk.shape
    q = qk.reshape(B, Lx, H * E)
    v = values.reshape(B, Lx, H * E)

    d_mats = _gram(q.astype(jnp.bfloat16))
    idx_pad, w_pad = _topk(d_mats)
    idx = idx_pad[0, :TOP_K]
    w = w_pad[:2, :]

    out = _aggregate(idx, w, v)
    return out.reshape(B, Lx, H, E), None


# speedup vs baseline: 1.0173x; 1.0173x over previous
import math

import jax
import jax.numpy as jnp
from jax.experimental import pallas as pl
from jax.experimental.pallas import tpu as pltpu

L = 2048
C = 16 * 64
T = 256
NB = L // T
ND = NB // 2 + 1
TOP_K = max(1, int(3 * math.log(L)))
NEG_INF = float("-inf")


def _gram_kernel(q_ref, d_ref):
    d = pl.program_id(1)
    acc = jnp.zeros((T, T), jnp.float32)
    for a in range(NB):
        qa = q_ref[0, pl.ds(a * T, T), :]
        ab = ((a + d) % NB) * T
        qb = q_ref[0, pl.ds(pl.multiple_of(ab, T), T), :]
        acc += jax.lax.dot_general(
            qa, qb,
            dimension_numbers=(((1,), (1,)), ((), ())),
            preferred_element_type=jnp.float32,
        )
    d_ref[0, 0] = acc


def _gram(q):
    return pl.pallas_call(
        _gram_kernel,
        grid=(2, ND),
        in_specs=[
            pl.BlockSpec((1, L, C), lambda b, d: (b, 0, 0)),
        ],
        out_specs=pl.BlockSpec((1, 1, T, T), lambda b, d: (b, d, 0, 0)),
        out_shape=jax.ShapeDtypeStruct((2, ND, T, T), jnp.float32),
    )(q)


def _topk_body(d_ref, idxv_ref, wv_ref):
    d04 = d_ref[...]
    parts = [d04] + [
        jnp.transpose(d04[:, k], (0, 2, 1)).reshape(2, 1, T, T)
        for k in (3, 2, 1)
    ]
    D = jnp.concatenate(parts, axis=1)
    E = jnp.concatenate([D, jnp.zeros_like(D)], axis=-1)
    u = jax.lax.broadcasted_iota(jnp.int32, E.shape, 2)
    for j in range(8):
        sh = 1 << j
        rolled = jnp.concatenate([E[..., sh:], E[..., :sh]], axis=-1)
        E = jnp.where((u & sh) != 0, rolled, E)
    corrp = jnp.sum(E, axis=2)
    nxt = jnp.roll(corrp, shift=-1, axis=1)
    mc = (corrp[:, :, :T] + nxt[:, :, T:]) * (1.0 / C)

    m = 0.5 * (mc[0] + mc[1])
    fi = (jax.lax.broadcasted_iota(jnp.int32, (NB, T), 0) * T
          + jax.lax.broadcasted_iota(jnp.int32, (NB, T), 1))
    lane = jax.lax.broadcasted_iota(jnp.int32, (8, 128), 1)
    row = jax.lax.broadcasted_iota(jnp.int32, (8, 128), 0)

    def body(i, carry):
        m, ivec, svec = carry
        val = jnp.max(m)
        idx = jnp.min(jnp.where(m == val, fi, jnp.int32(2 * L)))
        hit = fi == idx
        s0 = jnp.sum(jnp.where(hit, mc[0], 0.0))
        s1 = jnp.sum(jnp.where(hit, mc[1], 0.0))
        here = lane == i
        ivec = jnp.where((row == 0) & here, idx, ivec)
        svec = jnp.where((row == 0) & here, s0, svec)
        svec = jnp.where((row == 1) & here, s1, svec)
        m = jnp.where(hit, NEG_INF, m)
        return m, ivec, svec

    ivec = jnp.zeros((8, 128), jnp.int32)
    svec = jnp.zeros((8, 128), jnp.float32)
    m, ivec, svec = jax.lax.fori_loop(0, TOP_K, body, (m, ivec, svec))

    valid = lane < TOP_K
    x = jnp.where(valid, svec, NEG_INF)
    xmax = jnp.max(x, axis=1, keepdims=True)
    ex = jnp.where(valid, jnp.exp(x - xmax), 0.0)
    w = ex / jnp.sum(ex, axis=1, keepdims=True)

    idxv_ref[...] = ivec
    wv_ref[...] = w


def _fused_kernel(d_ref, v_ref, out_ref, idxv_ref, wv_ref):
    b = pl.program_id(0)
    c = pl.program_id(1)

    @pl.when((b == 0) & (c == 0))
    def _():
        _topk_body(d_ref, idxv_ref, wv_ref)

    d0 = idxv_ref[0, 0]
    w0 = wv_ref[b, 0]

    @pl.when(d0 == 0)
    def _():
        out_ref[0] = v_ref[0] * w0

    @pl.when(d0 != 0)
    def _():
        out_ref[0] = pltpu.roll(v_ref[0], -d0, axis=0) * w0

    for i in range(1, TOP_K):
        w = wv_ref[b, i]

        @pl.when(w != 0.0)
        def _(i=i, w=w):
            out_ref[0] += pltpu.roll(v_ref[0], -idxv_ref[0, i], axis=0) * w


def _fused(d_mats, v, cb=128):
    return pl.pallas_call(
        _fused_kernel,
        grid=(2, C // cb),
        in_specs=[
            pl.BlockSpec((2, ND, T, T), lambda b, c: (0, 0, 0, 0)),
            pl.BlockSpec((1, L, cb), lambda b, c: (b, 0, c)),
        ],
        out_specs=pl.BlockSpec((1, L, cb), lambda b, c: (b, 0, c)),
        out_shape=jax.ShapeDtypeStruct((2, L, C), jnp.float32),
        scratch_shapes=[
            pltpu.VMEM((8, 128), jnp.int32),
            pltpu.VMEM((8, 128), jnp.float32),
        ],
    )(d_mats, v)


@jax.jit
def kernel(qk, values):
    B, Lx, H, E = qk.shape
    q = qk.reshape(B, Lx, H * E)
    v = values.reshape(B, Lx, H * E)

    d_mats = _gram(q.astype(jnp.bfloat16))
    out = _fused(d_mats, v)
    return out.reshape(B, Lx, H, E), None


# speedup vs baseline: 1.2157x; 1.1950x over previous
import math

import jax
import jax.numpy as jnp
from jax.experimental import pallas as pl
from jax.experimental.pallas import tpu as pltpu

L = 2048
C = 16 * 64
T = 256
NB = L // T
ND = NB // 2 + 1
TOP_K = max(1, int(3 * math.log(L)))
NEG_INF = float("-inf")


def _gram_kernel(q_ref, d_ref, qb16_ref):
    d = pl.program_id(1)

    @pl.when(d == 0)
    def _():
        qb16_ref[...] = q_ref[0].astype(jnp.bfloat16)

    acc = jnp.zeros((T, T), jnp.float32)
    for a in range(NB):
        qa = qb16_ref[pl.ds(a * T, T), :]
        ab = ((a + d) % NB) * T
        qb = qb16_ref[pl.ds(pl.multiple_of(ab, T), T), :]
        acc += jax.lax.dot_general(
            qa, qb,
            dimension_numbers=(((1,), (1,)), ((), ())),
            preferred_element_type=jnp.float32,
        )
    d_ref[0, 0] = acc.astype(jnp.bfloat16)


def _gram(q):
    return pl.pallas_call(
        _gram_kernel,
        grid=(2, ND),
        in_specs=[
            pl.BlockSpec((1, L, C), lambda b, d: (b, 0, 0)),
        ],
        out_specs=pl.BlockSpec((1, 1, T, T), lambda b, d: (b, d, 0, 0)),
        out_shape=jax.ShapeDtypeStruct((2, ND, T, T), jnp.bfloat16),
        scratch_shapes=[pltpu.VMEM((L, C), jnp.bfloat16)],
    )(q)


def _topk_kernel(d_ref, idx_ref, w_ref):
    d04 = d_ref[...]
    parts = [d04] + [
        jnp.transpose(d04[:, k], (0, 2, 1)).reshape(2, 1, T, T)
        for k in (3, 2, 1)
    ]
    D = jnp.concatenate(parts, axis=1)
    E = jnp.concatenate([D, jnp.zeros_like(D)], axis=-1)
    u = jax.lax.broadcasted_iota(jnp.int32, E.shape, 2)
    for j in range(8):
        sh = 1 << j
        rolled = jnp.concatenate([E[..., sh:], E[..., :sh]], axis=-1)
        E = jnp.where((u & sh) != 0, rolled, E)
    corrp = jnp.sum(E.astype(jnp.float32), axis=2)
    nxt = jnp.roll(corrp, shift=-1, axis=1)
    mc = (corrp[:, :, :T] + nxt[:, :, T:]) * (1.0 / C)

    m = 0.5 * (mc[0] + mc[1])
    fi = (jax.lax.broadcasted_iota(jnp.int32, (NB, T), 0) * T
          + jax.lax.broadcasted_iota(jnp.int32, (NB, T), 1))
    lane = jax.lax.broadcasted_iota(jnp.int32, (8, 128), 1)
    row = jax.lax.broadcasted_iota(jnp.int32, (8, 128), 0)

    def body(i, carry):
        m, ivec, svec = carry
        val = jnp.max(m)
        idx = jnp.min(jnp.where(m == val, fi, jnp.int32(2 * L)))
        hit = fi == idx
        s0 = jnp.sum(jnp.where(hit, mc[0], 0.0))
        s1 = jnp.sum(jnp.where(hit, mc[1], 0.0))
        here = lane == i
        ivec = jnp.where((row == 0) & here, idx, ivec)
        svec = jnp.where((row == 0) & here, s0, svec)
        svec = jnp.where((row == 1) & here, s1, svec)
        m = jnp.where(hit, NEG_INF, m)
        return m, ivec, svec

    ivec = jnp.zeros((8, 128), jnp.int32)
    svec = jnp.zeros((8, 128), jnp.float32)
    m, ivec, svec = jax.lax.fori_loop(0, TOP_K, body, (m, ivec, svec))

    valid = lane < TOP_K
    x = jnp.where(valid, svec, NEG_INF)
    xmax = jnp.max(x, axis=1, keepdims=True)
    ex = jnp.where(valid, jnp.exp(x - xmax), 0.0)
    w = ex / jnp.sum(ex, axis=1, keepdims=True)

    idx_ref[...] = ivec
    w_ref[...] = w


def _topk(d_mats):
    return pl.pallas_call(
        _topk_kernel,
        out_shape=(
            jax.ShapeDtypeStruct((8, 128), jnp.int32),
            jax.ShapeDtypeStruct((8, 128), jnp.float32),
        ),
    )(d_mats)


def _agg_kernel(idx_ref, w_ref, v_ref, out_ref):
    b = pl.program_id(0)
    d0 = idx_ref[0]

    @pl.when(d0 == 0)
    def _():
        out_ref[0] = v_ref[0] * w_ref[b, 0]

    @pl.when(d0 != 0)
    def _():
        out_ref[0] = pltpu.roll(v_ref[0], -d0, axis=0) * w_ref[b, 0]

    for i in range(1, TOP_K):
        w = w_ref[b, i]

        @pl.when(w != 0.0)
        def _(i=i, w=w):
            out_ref[0] += pltpu.roll(v_ref[0], -idx_ref[i], axis=0) * w


def _aggregate(idx, w, v, cb=128):
    return pl.pallas_call(
        _agg_kernel,
        grid=(2, C // cb),
        in_specs=[
            pl.BlockSpec(memory_space=pltpu.SMEM),
            pl.BlockSpec(memory_space=pltpu.SMEM),
            pl.BlockSpec((1, L, cb), lambda b, c: (b, 0, c)),
        ],
        out_specs=pl.BlockSpec((1, L, cb), lambda b, c: (b, 0, c)),
        out_shape=jax.ShapeDtypeStruct((2, L, C), jnp.float32),
    )(idx, w, v)


@jax.jit
def kernel(qk, values):
    B, Lx, H, E = qk.shape
    q = qk.reshape(B, Lx, H * E)
    v = values.reshape(B, Lx, H * E)

    d_mats = _gram(q)
    idx_pad, w_pad = _topk(d_mats)
    idx = idx_pad[0, :TOP_K]
    w = w_pad[:2, :]

    out = _aggregate(idx, w, v)
    return out.reshape(B, Lx, H, E), None
